# baseline (speedup 1.0000x reference)
import functools

import jax
import jax.numpy as jnp
from jax import lax
from jax.experimental import pallas as pl
from jax.experimental.pallas import tpu as pltpu
from jax.experimental.pallas import tpu_sc as plsc

N = 10000
NR = 10240
IN_C = 128
HID = 256
OUT_C = 172
E_LOCAL = 300000
E_REMOTE = 20000
E_TOTAL = E_LOCAL + E_REMOTE + N
NTILES = 16
CHUNK = 128
NCHUNK = 168
SLAB = 12
NSLAB = NCHUNK // SLAB
E_PAD = NTILES * NCHUNK * CHUNK
RPT = NR // NTILES
RB = 512
HALF = NCHUNK // 2
NSLAB_E = HALF // SLAB

_f32 = jnp.float32
_mesh = plsc.VectorSubcoreMesh(core_axis_name="c", subcore_axis_name="s")
_params = pltpu.CompilerParams(needs_layout_passes=False)


@functools.partial(
    pl.kernel, mesh=_mesh,
    out_type=jax.ShapeDtypeStruct((2, NR), _f32),
    compiler_params=_params,
    scratch_types=[
        pltpu.VMEM((HALF, CHUNK), jnp.int32),
        pltpu.VMEM((NR,), _f32),
        pltpu.MemorySpace.VMEM_SHARED((NTILES, NR), _f32),
        pltpu.VMEM((NTILES, RPT), _f32),
        pltpu.VMEM((RPT,), _f32),
    ],
)
def _deg_sc(dsts32, deg_out, dstv, pdeg, stage, redbuf, res):
    c = lax.axis_index("c")
    s = lax.axis_index("s")
    pltpu.sync_copy(dsts32.at[c * NTILES + s], dstv)

    zeros16 = jnp.zeros((16,), _f32)
    ones16 = jnp.ones((16,), _f32)

    def _zero(i, _):
        pdeg[pl.ds(i * 16, 16)] = zeros16
        return ()
    lax.fori_loop(0, NR // 16, _zero, ())

    def _accum(k, _):
        r = k // (CHUNK // 16)
        i = k - r * (CHUNK // 16)
        idx = dstv[r, pl.ds(i * 16, 16)]
        plsc.addupdate_scatter(pdeg, [idx], ones16)
        return ()
    lax.fori_loop(0, HALF * (CHUNK // 16), _accum, ())

    pltpu.sync_copy(pdeg, stage.at[s])
    plsc.subcore_barrier()
    pltpu.sync_copy(stage.at[:, pl.ds(s * RPT, RPT)], redbuf)

    def _red(i, _):
        def _row(r, a):
            return a + redbuf[r, pl.ds(i * 16, 16)]
        res[pl.ds(i * 16, 16)] = lax.fori_loop(0, NTILES, _row, zeros16)
        return ()
    lax.fori_loop(0, RPT // 16, _red, ())
    pltpu.sync_copy(res, deg_out.at[c, pl.ds(s * RPT, RPT)])


def _edge_loop(g, acc, srcs4, dsts4, tid, srcv, dstv, rbs, sgs, sss, nslab):
    def slab_body(gs, _):
        pltpu.sync_copy(srcs4.at[tid, gs], srcv)
        pltpu.sync_copy(dsts4.at[tid, gs], dstv)
        pltpu.async_copy(g.at[srcv.at[0]], rbs[0], sgs[0])

        def pair(i, _):
            j = 2 * i
            for b in range(2):
                jj = j + b
                pltpu.make_async_copy(g.at[srcv.at[jj]], rbs[b],
                                      sgs[b]).wait()
                pltpu.async_copy(rbs[b], acc.at[dstv.at[jj]], sss[b],
                                 add=True)

                @pl.when(jj > 0)
                def _():
                    pltpu.make_async_copy(rbs[1 - b],
                                          acc.at[dstv.at[jj - 1]],
                                          sss[1 - b]).wait()
                nxt = jj + 1

                @pl.when(nxt < SLAB)
                def _():
                    pltpu.async_copy(g.at[srcv.at[nxt]], rbs[1 - b],
                                     sgs[1 - b])
            return ()
        lax.fori_loop(0, SLAB // 2, pair, ())
        pltpu.make_async_copy(rbs[1], acc.at[dstv.at[SLAB - 1]],
                              sss[1]).wait()
        return ()
    lax.fori_loop(0, nslab, slab_body, ())


def _zero_acc(acc, rb, s):
    zeros16 = jnp.zeros((16,), _f32)

    def _zb(i, _):
        r = i // (IN_C // 16)
        k = i - r * (IN_C // 16)
        rb[r, pl.ds(k * 16, 16)] = zeros16
        return ()
    lax.fori_loop(0, CHUNK * (IN_C // 16), _zb, ())

    def _cp(i, _):
        pltpu.sync_copy(rb, acc.at[pl.ds(s * RPT + i * CHUNK, CHUNK)])
        return ()
    lax.fori_loop(0, RPT // CHUNK, _cp, ())


@functools.partial(
    pl.kernel, mesh=_mesh,
    out_type=jax.ShapeDtypeStruct((2, NR, IN_C), _f32),
    compiler_params=_params,
    scratch_types=[
        pltpu.MemorySpace.VMEM_SHARED((NR, IN_C), _f32),
        pltpu.VMEM((SLAB, CHUNK), jnp.int32),
        pltpu.VMEM((SLAB, CHUNK), jnp.int32),
        pltpu.VMEM((CHUNK, IN_C), _f32),
        pltpu.VMEM((CHUNK, IN_C), _f32),
        pltpu.SemaphoreType.DMA,
        pltpu.SemaphoreType.DMA,
        pltpu.SemaphoreType.DMA,
        pltpu.SemaphoreType.DMA,
    ],
)
def _agg_e(g, srcs4, dsts4, out, acc, srcv, dstv, rb0, rb1,
           sg0, sg1, ss0, ss1):
    c = lax.axis_index("c")
    s = lax.axis_index("s")
    _zero_acc(acc, rb0, s)
    plsc.subcore_barrier()
    _edge_loop(g, acc, srcs4, dsts4, c * NTILES + s,
               srcv, dstv, (rb0, rb1), (sg0, sg1), (ss0, ss1), NSLAB_E)
    plsc.subcore_barrier()
    pltpu.sync_copy(acc.at[pl.ds(s * RPT, RPT)],
                    out.at[c, pl.ds(s * RPT, RPT)])


@functools.partial(
    pl.kernel, mesh=_mesh,
    out_type=(jax.ShapeDtypeStruct((NR, IN_C), _f32),
              jax.ShapeDtypeStruct((NR, IN_C), _f32)),
    compiler_params=_params,
    scratch_types=[
        pltpu.MemorySpace.VMEM_SHARED((NR, IN_C), _f32),
        pltpu.VMEM((SLAB, CHUNK), jnp.int32),
        pltpu.VMEM((SLAB, CHUNK), jnp.int32),
        pltpu.VMEM((CHUNK, IN_C), _f32),
        pltpu.VMEM((CHUNK, IN_C), _f32),
        pltpu.SemaphoreType.DMA,
        pltpu.SemaphoreType.DMA,
        pltpu.SemaphoreType.DMA,
        pltpu.SemaphoreType.DMA,
    ],
)
def _agg_c(g_lo, g_hi, srcs4, dsts4, out_lo, out_hi,
           acc, srcv, dstv, rb0, rb1, sg0, sg1, ss0, ss1):
    c = lax.axis_index("c")
    s = lax.axis_index("s")
    _zero_acc(acc, rb0, s)
    plsc.subcore_barrier()

    def run(g, out):
        _edge_loop(g, acc, srcs4, dsts4, s,
                   srcv, dstv, (rb0, rb1), (sg0, sg1), (ss0, ss1), NSLAB)
        plsc.subcore_barrier()
        pltpu.sync_copy(acc.at[pl.ds(s * RPT, RPT)],
                        out.at[pl.ds(s * RPT, RPT)])

    @pl.when(c == 0)
    def _():
        run(g_lo, out_lo)

    @pl.when(c == 1)
    def _():
        run(g_hi, out_hi)


def _tc_a_body(deg_ref, x_ref, dis_ref, u1_ref):
    i = pl.program_id(0)
    degsum = deg_ref[:, 0:1] + deg_ref[:, 1:2]
    rows = i * RB + lax.broadcasted_iota(jnp.int32, (RB, 1), 0)
    dis = jnp.where(rows < N, lax.rsqrt(jnp.maximum(degsum, 1.0)), 0.0)
    dis_ref[...] = jnp.broadcast_to(dis, (RB, IN_C))
    u1_ref[...] = x_ref[...] * dis


_tc_a = pl.pallas_call(
    _tc_a_body,
    grid=(NR // RB,),
    in_specs=[pl.BlockSpec((RB, 2), lambda i: (i, 0)),
              pl.BlockSpec((RB, IN_C), lambda i: (i, 0))],
    out_specs=[pl.BlockSpec((RB, IN_C), lambda i: (i, 0)),
               pl.BlockSpec((RB, IN_C), lambda i: (i, 0))],
    out_shape=[jax.ShapeDtypeStruct((NR, IN_C), _f32),
               jax.ShapeDtypeStruct((NR, IN_C), _f32)],
)


def _tc_b_body(v1_ref, dis_ref, w1_ref, b1_ref, w2_ref, ulo_ref, uhi_ref):
    dis = dis_ref[:, 0:1]
    v1 = v1_ref[0] + v1_ref[1]
    h1 = jnp.dot(v1 * dis, w1_ref[...], preferred_element_type=_f32)
    h1 = jnp.maximum(h1 + b1_ref[...], 0.0)
    p2 = jnp.dot(h1, w2_ref[...], preferred_element_type=_f32)
    u2 = p2 * dis
    ulo_ref[...] = u2[:, :HID // 2]
    uhi_ref[...] = u2[:, HID // 2:]


_tc_b = pl.pallas_call(
    _tc_b_body,
    grid=(NR // RB,),
    in_specs=[pl.BlockSpec((2, RB, IN_C), lambda i: (0, i, 0)),
              pl.BlockSpec((RB, IN_C), lambda i: (i, 0)),
              pl.BlockSpec((IN_C, HID), lambda i: (0, 0)),
              pl.BlockSpec((1, HID), lambda i: (0, 0)),
              pl.BlockSpec((HID, HID), lambda i: (0, 0))],
    out_specs=[pl.BlockSpec((RB, HID // 2), lambda i: (i, 0)),
               pl.BlockSpec((RB, HID // 2), lambda i: (i, 0))],
    out_shape=[jax.ShapeDtypeStruct((NR, HID // 2), _f32),
               jax.ShapeDtypeStruct((NR, HID // 2), _f32)],
)


def _tc_c_body(vlo_ref, vhi_ref, dis_ref, b2_ref, ulo_ref, uhi_ref):
    dis = dis_ref[:, 0:1]
    v2 = jnp.concatenate([vlo_ref[...], vhi_ref[...]], axis=1)
    h2 = jnp.maximum(v2 * dis + b2_ref[...], 0.0)
    u3 = h2 * dis
    ulo_ref[...] = u3[:, :HID // 2]
    uhi_ref[...] = u3[:, HID // 2:]


_tc_c = pl.pallas_call(
    _tc_c_body,
    grid=(NR // RB,),
    in_specs=[pl.BlockSpec((RB, HID // 2), lambda i: (i, 0)),
              pl.BlockSpec((RB, HID // 2), lambda i: (i, 0)),
              pl.BlockSpec((RB, IN_C), lambda i: (i, 0)),
              pl.BlockSpec((1, HID), lambda i: (0, 0))],
    out_specs=[pl.BlockSpec((RB, HID // 2), lambda i: (i, 0)),
               pl.BlockSpec((RB, HID // 2), lambda i: (i, 0))],
    out_shape=[jax.ShapeDtypeStruct((NR, HID // 2), _f32),
               jax.ShapeDtypeStruct((NR, HID // 2), _f32)],
)


def _tc_d_body(vlo_ref, vhi_ref, dis_ref, w3_ref, b3_ref, out_ref):
    dis = dis_ref[:, 0:1]
    v3 = jnp.concatenate([vlo_ref[...], vhi_ref[...]], axis=1)
    z = jnp.dot(v3 * dis, w3_ref[...], preferred_element_type=_f32)
    z = z + b3_ref[...]
    m = jnp.max(z, axis=1, keepdims=True)
    e = jnp.exp(z - m)
    lse = jnp.log(jnp.sum(e, axis=1, keepdims=True)) + m
    out_ref[...] = z - lse


_tc_d = pl.pallas_call(
    _tc_d_body,
    grid=(NR // RB,),
    in_specs=[pl.BlockSpec((RB, HID // 2), lambda i: (i, 0)),
              pl.BlockSpec((RB, HID // 2), lambda i: (i, 0)),
              pl.BlockSpec((RB, IN_C), lambda i: (i, 0)),
              pl.BlockSpec((HID, OUT_C), lambda i: (0, 0)),
              pl.BlockSpec((1, OUT_C), lambda i: (0, 0))],
    out_specs=pl.BlockSpec((RB, OUT_C), lambda i: (i, 0)),
    out_shape=jax.ShapeDtypeStruct((NR, OUT_C), _f32),
)


def kernel(x, local_edges_list, remote_edges_list, W1, b1, W2, b2, W3, b3):
    loops = jnp.arange(N, dtype=jnp.int32)
    src = jnp.concatenate([local_edges_list[0], remote_edges_list[0], loops])
    dst = jnp.concatenate([local_edges_list[1], remote_edges_list[1], loops])
    src = jnp.pad(src, (0, E_PAD - E_TOTAL), constant_values=N)
    dst = jnp.pad(dst, (0, E_PAD - E_TOTAL), constant_values=N)
    srcs4 = src.reshape(NTILES, NSLAB, SLAB, CHUNK)
    dsts4 = dst.reshape(NTILES, NSLAB, SLAB, CHUNK)
    srcs32 = src.reshape(NSLAB_E, 2 * NTILES, SLAB, CHUNK).transpose(
        1, 0, 2, 3)
    dsts32 = dst.reshape(NSLAB_E, 2 * NTILES, SLAB, CHUNK).transpose(
        1, 0, 2, 3)
    dsts32d = dst.reshape(2 * NTILES, HALF, CHUNK)

    xp = jnp.pad(x, ((0, NR - N), (0, 0)))
    b1r = b1.reshape(1, HID)
    b2r = b2.reshape(1, HID)
    b3r = b3.reshape(1, OUT_C)

    deg2 = _deg_sc(dsts32d)
    dis, u1 = _tc_a(deg2.T, xp)
    v1p = _agg_e(u1, srcs32, dsts32)
    u2lo, u2hi = _tc_b(v1p, dis, W1, b1r, W2)
    v2lo, v2hi = _agg_c(u2lo, u2hi, srcs4, dsts4)
    u3lo, u3hi = _tc_c(v2lo, v2hi, dis, b2r)
    v3lo, v3hi = _agg_c(u3lo, u3hi, srcs4, dsts4)
    out = _tc_d(v3lo, v3hi, dis, W3, b3r)
    return out[:N]


# speedup vs baseline: 1.0261x; 1.0261x over previous
import functools

import jax
import jax.numpy as jnp
from jax import lax
from jax.experimental import pallas as pl
from jax.experimental.pallas import tpu as pltpu
from jax.experimental.pallas import tpu_sc as plsc

N = 10000
NR = 10240
IN_C = 128
HID = 256
OUT_C = 172
E_LOCAL = 300000
E_REMOTE = 20000
E_TOTAL = E_LOCAL + E_REMOTE + N
NTILES = 16
CHUNK = 64
NCHUNK = 336
SLAB = 24
NSLAB = NCHUNK // SLAB
E_PAD = NTILES * NCHUNK * CHUNK
RPT = NR // NTILES
RB = 512
HALF = NCHUNK // 2
NSLAB_E = HALF // SLAB

_f32 = jnp.float32
_mesh = plsc.VectorSubcoreMesh(core_axis_name="c", subcore_axis_name="s")
_params = pltpu.CompilerParams(needs_layout_passes=False)


@functools.partial(
    pl.kernel, mesh=_mesh,
    out_type=jax.ShapeDtypeStruct((2, NR), _f32),
    compiler_params=_params,
    scratch_types=[
        pltpu.VMEM((HALF, CHUNK), jnp.int32),
        pltpu.VMEM((NR,), _f32),
        pltpu.MemorySpace.VMEM_SHARED((NTILES, NR), _f32),
        pltpu.VMEM((NTILES, RPT), _f32),
        pltpu.VMEM((RPT,), _f32),
    ],
)
def _deg_sc(dsts32, deg_out, dstv, pdeg, stage, redbuf, res):
    c = lax.axis_index("c")
    s = lax.axis_index("s")
    pltpu.sync_copy(dsts32.at[c * NTILES + s], dstv)

    zeros16 = jnp.zeros((16,), _f32)
    ones16 = jnp.ones((16,), _f32)

    def _zero(i, _):
        pdeg[pl.ds(i * 16, 16)] = zeros16
        return ()
    lax.fori_loop(0, NR // 16, _zero, ())

    def _accum(k, _):
        r = k // (CHUNK // 16)
        i = k - r * (CHUNK // 16)
        idx = dstv[r, pl.ds(i * 16, 16)]
        plsc.addupdate_scatter(pdeg, [idx], ones16)
        return ()
    lax.fori_loop(0, HALF * (CHUNK // 16), _accum, ())

    pltpu.sync_copy(pdeg, stage.at[s])
    plsc.subcore_barrier()
    pltpu.sync_copy(stage.at[:, pl.ds(s * RPT, RPT)], redbuf)

    def _red(i, _):
        def _row(r, a):
            return a + redbuf[r, pl.ds(i * 16, 16)]
        res[pl.ds(i * 16, 16)] = lax.fori_loop(0, NTILES, _row, zeros16)
        return ()
    lax.fori_loop(0, RPT // 16, _red, ())
    pltpu.sync_copy(res, deg_out.at[c, pl.ds(s * RPT, RPT)])


def _edge_loop(g, acc, srcs4, dsts4, tid, srcv, dstv, rbs, sgs, sss, nslab):
    def slab_body(gs, _):
        pltpu.sync_copy(srcs4.at[tid, gs], srcv)
        pltpu.sync_copy(dsts4.at[tid, gs], dstv)
        for b in range(3):
            pltpu.async_copy(g.at[srcv.at[b]], rbs[b], sgs[b])

        def quad(i, _):
            j0 = 4 * i
            for b in range(4):
                j = j0 + b
                pltpu.make_async_copy(g.at[srcv.at[j]], rbs[b],
                                      sgs[b]).wait()
                pltpu.async_copy(rbs[b], acc.at[dstv.at[j]], sss[b],
                                 add=True)
                bn = (b + 3) % 4

                @pl.when(j + 3 < SLAB)
                def _():
                    @pl.when(j >= 1)
                    def _():
                        pltpu.make_async_copy(rbs[bn],
                                              acc.at[dstv.at[j - 1]],
                                              sss[bn]).wait()
                    pltpu.async_copy(g.at[srcv.at[j + 3]], rbs[bn],
                                     sgs[bn])
            return ()
        lax.fori_loop(0, SLAB // 4, quad, ())
        for b in range(4):
            pltpu.make_async_copy(rbs[b], acc.at[dstv.at[SLAB - 4 + b]],
                                  sss[b]).wait()
        return ()
    lax.fori_loop(0, nslab, slab_body, ())


def _zero_acc(acc, rb, s):
    zeros16 = jnp.zeros((16,), _f32)

    def _zb(i, _):
        r = i // (IN_C // 16)
        k = i - r * (IN_C // 16)
        rb[r, pl.ds(k * 16, 16)] = zeros16
        return ()
    lax.fori_loop(0, CHUNK * (IN_C // 16), _zb, ())

    def _cp(i, _):
        pltpu.sync_copy(rb, acc.at[pl.ds(s * RPT + i * CHUNK, CHUNK)])
        return ()
    lax.fori_loop(0, RPT // CHUNK, _cp, ())


@functools.partial(
    pl.kernel, mesh=_mesh,
    out_type=jax.ShapeDtypeStruct((2, NR, IN_C), _f32),
    compiler_params=_params,
    scratch_types=[
        pltpu.MemorySpace.VMEM_SHARED((NR, IN_C), _f32),
        pltpu.VMEM((SLAB, CHUNK), jnp.int32),
        pltpu.VMEM((SLAB, CHUNK), jnp.int32),
        pltpu.VMEM((CHUNK, IN_C), _f32),
        pltpu.VMEM((CHUNK, IN_C), _f32),
        pltpu.VMEM((CHUNK, IN_C), _f32),
        pltpu.VMEM((CHUNK, IN_C), _f32),
        pltpu.SemaphoreType.DMA,
        pltpu.SemaphoreType.DMA,
        pltpu.SemaphoreType.DMA,
        pltpu.SemaphoreType.DMA,
        pltpu.SemaphoreType.DMA,
        pltpu.SemaphoreType.DMA,
        pltpu.SemaphoreType.DMA,
        pltpu.SemaphoreType.DMA,
    ],
)
def _agg_e(g, srcs4, dsts4, out, acc, srcv, dstv, rb0, rb1, rb2, rb3,
           sg0, sg1, sg2, sg3, ss0, ss1, ss2, ss3):
    c = lax.axis_index("c")
    s = lax.axis_index("s")
    _zero_acc(acc, rb0, s)
    plsc.subcore_barrier()
    _edge_loop(g, acc, srcs4, dsts4, c * NTILES + s, srcv, dstv,
               (rb0, rb1, rb2, rb3), (sg0, sg1, sg2, sg3),
               (ss0, ss1, ss2, ss3), NSLAB_E)
    plsc.subcore_barrier()
    pltpu.sync_copy(acc.at[pl.ds(s * RPT, RPT)],
                    out.at[c, pl.ds(s * RPT, RPT)])


@functools.partial(
    pl.kernel, mesh=_mesh,
    out_type=(jax.ShapeDtypeStruct((NR, IN_C), _f32),
              jax.ShapeDtypeStruct((NR, IN_C), _f32)),
    compiler_params=_params,
    scratch_types=[
        pltpu.MemorySpace.VMEM_SHARED((NR, IN_C), _f32),
        pltpu.VMEM((SLAB, CHUNK), jnp.int32),
        pltpu.VMEM((SLAB, CHUNK), jnp.int32),
        pltpu.VMEM((CHUNK, IN_C), _f32),
        pltpu.VMEM((CHUNK, IN_C), _f32),
        pltpu.VMEM((CHUNK, IN_C), _f32),
        pltpu.VMEM((CHUNK, IN_C), _f32),
        pltpu.SemaphoreType.DMA,
        pltpu.SemaphoreType.DMA,
        pltpu.SemaphoreType.DMA,
        pltpu.SemaphoreType.DMA,
        pltpu.SemaphoreType.DMA,
        pltpu.SemaphoreType.DMA,
        pltpu.SemaphoreType.DMA,
        pltpu.SemaphoreType.DMA,
    ],
)
def _agg_c(g_lo, g_hi, srcs4, dsts4, out_lo, out_hi,
           acc, srcv, dstv, rb0, rb1, rb2, rb3,
           sg0, sg1, sg2, sg3, ss0, ss1, ss2, ss3):
    c = lax.axis_index("c")
    s = lax.axis_index("s")
    _zero_acc(acc, rb0, s)
    plsc.subcore_barrier()

    def run(g, out):
        _edge_loop(g, acc, srcs4, dsts4, s, srcv, dstv,
                   (rb0, rb1, rb2, rb3), (sg0, sg1, sg2, sg3),
                   (ss0, ss1, ss2, ss3), NSLAB)
        plsc.subcore_barrier()
        pltpu.sync_copy(acc.at[pl.ds(s * RPT, RPT)],
                        out.at[pl.ds(s * RPT, RPT)])

    @pl.when(c == 0)
    def _():
        run(g_lo, out_lo)

    @pl.when(c == 1)
    def _():
        run(g_hi, out_hi)


def _tc_a_body(deg_ref, x_ref, dis_ref, u1_ref):
    i = pl.program_id(0)
    degsum = deg_ref[:, 0:1] + deg_ref[:, 1:2]
    rows = i * RB + lax.broadcasted_iota(jnp.int32, (RB, 1), 0)
    dis = jnp.where(rows < N, lax.rsqrt(jnp.maximum(degsum, 1.0)), 0.0)
    dis_ref[...] = jnp.broadcast_to(dis, (RB, IN_C))
    u1_ref[...] = x_ref[...] * dis


_tc_a = pl.pallas_call(
    _tc_a_body,
    grid=(NR // RB,),
    in_specs=[pl.BlockSpec((RB, 2), lambda i: (i, 0)),
              pl.BlockSpec((RB, IN_C), lambda i: (i, 0))],
    out_specs=[pl.BlockSpec((RB, IN_C), lambda i: (i, 0)),
               pl.BlockSpec((RB, IN_C), lambda i: (i, 0))],
    out_shape=[jax.ShapeDtypeStruct((NR, IN_C), _f32),
               jax.ShapeDtypeStruct((NR, IN_C), _f32)],
)


def _tc_b_body(v1_ref, dis_ref, w1_ref, b1_ref, w2_ref, ulo_ref, uhi_ref):
    dis = dis_ref[:, 0:1]
    v1 = v1_ref[0] + v1_ref[1]
    h1 = jnp.dot(v1 * dis, w1_ref[...], preferred_element_type=_f32)
    h1 = jnp.maximum(h1 + b1_ref[...], 0.0)
    p2 = jnp.dot(h1, w2_ref[...], preferred_element_type=_f32)
    u2 = p2 * dis
    ulo_ref[...] = u2[:, :HID // 2]
    uhi_ref[...] = u2[:, HID // 2:]


_tc_b = pl.pallas_call(
    _tc_b_body,
    grid=(NR // RB,),
    in_specs=[pl.BlockSpec((2, RB, IN_C), lambda i: (0, i, 0)),
              pl.BlockSpec((RB, IN_C), lambda i: (i, 0)),
              pl.BlockSpec((IN_C, HID), lambda i: (0, 0)),
              pl.BlockSpec((1, HID), lambda i: (0, 0)),
              pl.BlockSpec((HID, HID), lambda i: (0, 0))],
    out_specs=[pl.BlockSpec((RB, HID // 2), lambda i: (i, 0)),
               pl.BlockSpec((RB, HID // 2), lambda i: (i, 0))],
    out_shape=[jax.ShapeDtypeStruct((NR, HID // 2), _f32),
               jax.ShapeDtypeStruct((NR, HID // 2), _f32)],
)


def _tc_c_body(vlo_ref, vhi_ref, dis_ref, b2_ref, ulo_ref, uhi_ref):
    dis = dis_ref[:, 0:1]
    v2 = jnp.concatenate([vlo_ref[...], vhi_ref[...]], axis=1)
    h2 = jnp.maximum(v2 * dis + b2_ref[...], 0.0)
    u3 = h2 * dis
    ulo_ref[...] = u3[:, :HID // 2]
    uhi_ref[...] = u3[:, HID // 2:]


_tc_c = pl.pallas_call(
    _tc_c_body,
    grid=(NR // RB,),
    in_specs=[pl.BlockSpec((RB, HID // 2), lambda i: (i, 0)),
              pl.BlockSpec((RB, HID // 2), lambda i: (i, 0)),
              pl.BlockSpec((RB, IN_C), lambda i: (i, 0)),
              pl.BlockSpec((1, HID), lambda i: (0, 0))],
    out_specs=[pl.BlockSpec((RB, HID // 2), lambda i: (i, 0)),
               pl.BlockSpec((RB, HID // 2), lambda i: (i, 0))],
    out_shape=[jax.ShapeDtypeStruct((NR, HID // 2), _f32),
               jax.ShapeDtypeStruct((NR, HID // 2), _f32)],
)


def _tc_d_body(vlo_ref, vhi_ref, dis_ref, w3_ref, b3_ref, out_ref):
    dis = dis_ref[:, 0:1]
    v3 = jnp.concatenate([vlo_ref[...], vhi_ref[...]], axis=1)
    z = jnp.dot(v3 * dis, w3_ref[...], preferred_element_type=_f32)
    z = z + b3_ref[...]
    m = jnp.max(z, axis=1, keepdims=True)
    e = jnp.exp(z - m)
    lse = jnp.log(jnp.sum(e, axis=1, keepdims=True)) + m
    out_ref[...] = z - lse


_tc_d = pl.pallas_call(
    _tc_d_body,
    grid=(NR // RB,),
    in_specs=[pl.BlockSpec((RB, HID // 2), lambda i: (i, 0)),
              pl.BlockSpec((RB, HID // 2), lambda i: (i, 0)),
              pl.BlockSpec((RB, IN_C), lambda i: (i, 0)),
              pl.BlockSpec((HID, OUT_C), lambda i: (0, 0)),
              pl.BlockSpec((1, OUT_C), lambda i: (0, 0))],
    out_specs=pl.BlockSpec((RB, OUT_C), lambda i: (i, 0)),
    out_shape=jax.ShapeDtypeStruct((NR, OUT_C), _f32),
)


def kernel(x, local_edges_list, remote_edges_list, W1, b1, W2, b2, W3, b3):
    loops = jnp.arange(N, dtype=jnp.int32)
    src = jnp.concatenate([local_edges_list[0], remote_edges_list[0], loops])
    dst = jnp.concatenate([local_edges_list[1], remote_edges_list[1], loops])
    src = jnp.pad(src, (0, E_PAD - E_TOTAL), constant_values=N)
    dst = jnp.pad(dst, (0, E_PAD - E_TOTAL), constant_values=N)
    srcs4 = src.reshape(NTILES, NSLAB, SLAB, CHUNK)
    dsts4 = dst.reshape(NTILES, NSLAB, SLAB, CHUNK)
    srcs32 = src.reshape(NSLAB_E, 2 * NTILES, SLAB, CHUNK).transpose(
        1, 0, 2, 3)
    dsts32 = dst.reshape(NSLAB_E, 2 * NTILES, SLAB, CHUNK).transpose(
        1, 0, 2, 3)
    dsts32d = dst.reshape(2 * NTILES, HALF, CHUNK)

    xp = jnp.pad(x, ((0, NR - N), (0, 0)))
    b1r = b1.reshape(1, HID)
    b2r = b2.reshape(1, HID)
    b3r = b3.reshape(1, OUT_C)

    deg2 = _deg_sc(dsts32d)
    dis, u1 = _tc_a(deg2.T, xp)
    v1p = _agg_e(u1, srcs32, dsts32)
    u2lo, u2hi = _tc_b(v1p, dis, W1, b1r, W2)
    v2lo, v2hi = _agg_c(u2lo, u2hi, srcs4, dsts4)
    u3lo, u3hi = _tc_c(v2lo, v2hi, dis, b2r)
    v3lo, v3hi = _agg_c(u3lo, u3hi, srcs4, dsts4)
    out = _tc_d(v3lo, v3hi, dis, W3, b3r)
    return out[:N]


# speedup vs baseline: 1.4389x; 1.4023x over previous
import functools

import jax
import jax.numpy as jnp
from jax import lax
from jax.experimental import pallas as pl
from jax.experimental.pallas import tpu as pltpu
from jax.experimental.pallas import tpu_sc as plsc

N = 10000
NR = 10240
IN_C = 128
HID = 256
OUT_C = 172
E_LOCAL = 300000
E_REMOTE = 20000
E_TOTAL = E_LOCAL + E_REMOTE
NTILES = 16
CHUNK = 64
NCHUNK = 320
SLAB = 16
NSLAB = NCHUNK // SLAB
E_PAD = NTILES * NCHUNK * CHUNK
RPT = NR // NTILES
RB = 512
HALF = NCHUNK // 2
NSLAB_E = HALF // SLAB

_f32 = jnp.float32
_mesh = plsc.VectorSubcoreMesh(core_axis_name="c", subcore_axis_name="s")
_params = pltpu.CompilerParams(needs_layout_passes=False)


@functools.partial(
    pl.kernel, mesh=_mesh,
    out_type=jax.ShapeDtypeStruct((2, NR), _f32),
    compiler_params=_params,
    scratch_types=[
        pltpu.VMEM((HALF, CHUNK), jnp.int32),
        pltpu.VMEM((NR,), _f32),
        pltpu.MemorySpace.VMEM_SHARED((NTILES, NR), _f32),
        pltpu.VMEM((NTILES, RPT), _f32),
        pltpu.VMEM((RPT,), _f32),
    ],
)
def _deg_sc(dsts32, deg_out, dstv, pdeg, stage, redbuf, res):
    c = lax.axis_index("c")
    s = lax.axis_index("s")
    pltpu.sync_copy(dsts32.at[c * NTILES + s], dstv)

    zeros16 = jnp.zeros((16,), _f32)
    ones16 = jnp.ones((16,), _f32)

    def _zero(i, _):
        pdeg[pl.ds(i * 16, 16)] = zeros16
        return ()
    lax.fori_loop(0, NR // 16, _zero, ())

    def _accum(k, _):
        r = k // (CHUNK // 16)
        i = k - r * (CHUNK // 16)
        idx = dstv[r, pl.ds(i * 16, 16)]
        plsc.addupdate_scatter(pdeg, [idx], ones16)
        return ()
    lax.fori_loop(0, HALF * (CHUNK // 16), _accum, ())

    pltpu.sync_copy(pdeg, stage.at[s])
    plsc.subcore_barrier()
    pltpu.sync_copy(stage.at[:, pl.ds(s * RPT, RPT)], redbuf)

    def _red(i, _):
        def _row(r, a):
            return a + redbuf[r, pl.ds(i * 16, 16)]
        res[pl.ds(i * 16, 16)] = lax.fori_loop(0, NTILES, _row, zeros16)
        return ()
    lax.fori_loop(0, RPT // 16, _red, ())
    pltpu.sync_copy(res, deg_out.at[c, pl.ds(s * RPT, RPT)])


def _edge_loop(g, acc, srcs4, dsts4, tid, srcv, dstv, rbs, sgs, sss, nslab):
    def slab_body(gs, _):
        pltpu.sync_copy(srcs4.at[tid, gs], srcv)
        pltpu.sync_copy(dsts4.at[tid, gs], dstv)
        for b in range(3):
            pltpu.async_copy(g.at[srcv.at[b]], rbs[b], sgs[b])

        def quad(i, _):
            j0 = 4 * i
            for b in range(4):
                j = j0 + b
                pltpu.make_async_copy(g.at[srcv.at[j]], rbs[b],
                                      sgs[b]).wait()
                pltpu.async_copy(rbs[b], acc.at[dstv.at[j]], sss[b],
                                 add=True)
                bn = (b + 3) % 4

                @pl.when(j + 3 < SLAB)
                def _():
                    @pl.when(j >= 1)
                    def _():
                        pltpu.make_async_copy(rbs[bn],
                                              acc.at[dstv.at[j - 1]],
                                              sss[bn]).wait()
                    pltpu.async_copy(g.at[srcv.at[j + 3]], rbs[bn],
                                     sgs[bn])
            return ()
        lax.fori_loop(0, SLAB // 4, quad, ())
        for b in range(4):
            pltpu.make_async_copy(rbs[b], acc.at[dstv.at[SLAB - 4 + b]],
                                  sss[b]).wait()
        return ()
    lax.fori_loop(0, nslab, slab_body, ())


def _zero_acc(acc, rb, s):
    zeros16 = jnp.zeros((16,), _f32)

    def _zb(i, _):
        r = i // (IN_C // 16)
        k = i - r * (IN_C // 16)
        rb[r, pl.ds(k * 16, 16)] = zeros16
        return ()
    lax.fori_loop(0, CHUNK * (IN_C // 16), _zb, ())

    def _cp(i, _):
        pltpu.sync_copy(rb, acc.at[pl.ds(s * RPT + i * CHUNK, CHUNK)])
        return ()
    lax.fori_loop(0, RPT // CHUNK, _cp, ())


@functools.partial(
    pl.kernel, mesh=_mesh,
    out_type=jax.ShapeDtypeStruct((2, NR, IN_C), _f32),
    compiler_params=_params,
    scratch_types=[
        pltpu.MemorySpace.VMEM_SHARED((NR, IN_C), _f32),
        pltpu.VMEM((SLAB, CHUNK), jnp.int32),
        pltpu.VMEM((SLAB, CHUNK), jnp.int32),
        pltpu.VMEM((CHUNK, IN_C), _f32),
        pltpu.VMEM((CHUNK, IN_C), _f32),
        pltpu.VMEM((CHUNK, IN_C), _f32),
        pltpu.VMEM((CHUNK, IN_C), _f32),
        pltpu.SemaphoreType.DMA,
        pltpu.SemaphoreType.DMA,
        pltpu.SemaphoreType.DMA,
        pltpu.SemaphoreType.DMA,
        pltpu.SemaphoreType.DMA,
        pltpu.SemaphoreType.DMA,
        pltpu.SemaphoreType.DMA,
        pltpu.SemaphoreType.DMA,
    ],
)
def _agg_e(g_a, g_b, srcs4, dsts4, out, acc, srcv, dstv, rb0, rb1, rb2, rb3,
           sg0, sg1, sg2, sg3, ss0, ss1, ss2, ss3):
    c = lax.axis_index("c")
    s = lax.axis_index("s")
    _zero_acc(acc, rb0, s)
    plsc.subcore_barrier()

    def run(g):
        _edge_loop(g, acc, srcs4, dsts4, c * NTILES + s, srcv, dstv,
                   (rb0, rb1, rb2, rb3), (sg0, sg1, sg2, sg3),
                   (ss0, ss1, ss2, ss3), NSLAB_E)

    @pl.when(c == 0)
    def _():
        run(g_a)

    @pl.when(c == 1)
    def _():
        run(g_b)
    plsc.subcore_barrier()
    pltpu.sync_copy(acc.at[pl.ds(s * RPT, RPT)],
                    out.at[c, pl.ds(s * RPT, RPT)])


@functools.partial(
    pl.kernel, mesh=_mesh,
    out_type=(jax.ShapeDtypeStruct((NR, IN_C), _f32),
              jax.ShapeDtypeStruct((NR, IN_C), _f32)),
    compiler_params=_params,
    scratch_types=[
        pltpu.MemorySpace.VMEM_SHARED((NR, IN_C), _f32),
        pltpu.VMEM((SLAB, CHUNK), jnp.int32),
        pltpu.VMEM((SLAB, CHUNK), jnp.int32),
        pltpu.VMEM((CHUNK, IN_C), _f32),
        pltpu.VMEM((CHUNK, IN_C), _f32),
        pltpu.VMEM((CHUNK, IN_C), _f32),
        pltpu.VMEM((CHUNK, IN_C), _f32),
        pltpu.SemaphoreType.DMA,
        pltpu.SemaphoreType.DMA,
        pltpu.SemaphoreType.DMA,
        pltpu.SemaphoreType.DMA,
        pltpu.SemaphoreType.DMA,
        pltpu.SemaphoreType.DMA,
        pltpu.SemaphoreType.DMA,
        pltpu.SemaphoreType.DMA,
    ],
)
def _agg_c(g_lo, g_hi, srcs4, dsts4, out_lo, out_hi,
           acc, srcv, dstv, rb0, rb1, rb2, rb3,
           sg0, sg1, sg2, sg3, ss0, ss1, ss2, ss3):
    c = lax.axis_index("c")
    s = lax.axis_index("s")
    _zero_acc(acc, rb0, s)
    plsc.subcore_barrier()

    def run(g, out):
        _edge_loop(g, acc, srcs4, dsts4, s, srcv, dstv,
                   (rb0, rb1, rb2, rb3), (sg0, sg1, sg2, sg3),
                   (ss0, ss1, ss2, ss3), NSLAB)
        plsc.subcore_barrier()
        pltpu.sync_copy(acc.at[pl.ds(s * RPT, RPT)],
                        out.at[pl.ds(s * RPT, RPT)])

    @pl.when(c == 0)
    def _():
        run(g_lo, out_lo)

    @pl.when(c == 1)
    def _():
        run(g_hi, out_hi)


def _tc_a_body(deg_ref, x_ref, dis_ref, u1a_ref, u1b_ref):
    i = pl.program_id(0)
    degsum = deg_ref[:, 0:1] + deg_ref[:, 1:2] + 1.0
    rows = i * RB + lax.broadcasted_iota(jnp.int32, (RB, 1), 0)
    dis = jnp.where(rows < N, lax.rsqrt(degsum), 0.0)
    dis_ref[...] = jnp.broadcast_to(dis, (RB, IN_C))
    u1 = x_ref[...] * dis
    u1a_ref[...] = u1
    u1b_ref[...] = u1


_tc_a = pl.pallas_call(
    _tc_a_body,
    grid=(NR // RB,),
    in_specs=[pl.BlockSpec((RB, 2), lambda i: (i, 0)),
              pl.BlockSpec((RB, IN_C), lambda i: (i, 0))],
    out_specs=[pl.BlockSpec((RB, IN_C), lambda i: (i, 0)),
               pl.BlockSpec((RB, IN_C), lambda i: (i, 0)),
               pl.BlockSpec((RB, IN_C), lambda i: (i, 0))],
    out_shape=[jax.ShapeDtypeStruct((NR, IN_C), _f32),
               jax.ShapeDtypeStruct((NR, IN_C), _f32),
               jax.ShapeDtypeStruct((NR, IN_C), _f32)],
)


def _tc_b_body(v1_ref, u1_ref, dis_ref, w1_ref, b1_ref, w2_ref,
               ulo_ref, uhi_ref):
    dis = dis_ref[:, 0:1]
    v1 = v1_ref[0] + v1_ref[1] + u1_ref[...]
    h1 = jnp.dot(v1 * dis, w1_ref[...], preferred_element_type=_f32)
    h1 = jnp.maximum(h1 + b1_ref[...], 0.0)
    p2 = jnp.dot(h1, w2_ref[...], preferred_element_type=_f32)
    u2 = p2 * dis
    ulo_ref[...] = u2[:, :HID // 2]
    uhi_ref[...] = u2[:, HID // 2:]


_tc_b = pl.pallas_call(
    _tc_b_body,
    grid=(NR // RB,),
    in_specs=[pl.BlockSpec((2, RB, IN_C), lambda i: (0, i, 0)),
              pl.BlockSpec((RB, IN_C), lambda i: (i, 0)),
              pl.BlockSpec((RB, IN_C), lambda i: (i, 0)),
              pl.BlockSpec((IN_C, HID), lambda i: (0, 0)),
              pl.BlockSpec((1, HID), lambda i: (0, 0)),
              pl.BlockSpec((HID, HID), lambda i: (0, 0))],
    out_specs=[pl.BlockSpec((RB, HID // 2), lambda i: (i, 0)),
               pl.BlockSpec((RB, HID // 2), lambda i: (i, 0))],
    out_shape=[jax.ShapeDtypeStruct((NR, HID // 2), _f32),
               jax.ShapeDtypeStruct((NR, HID // 2), _f32)],
)


def _tc_c_body(vlo_ref, vhi_ref, u2lo_ref, u2hi_ref, dis_ref, b2_ref,
               ulo_ref, uhi_ref):
    dis = dis_ref[:, 0:1]
    v2 = jnp.concatenate([vlo_ref[...] + u2lo_ref[...],
                          vhi_ref[...] + u2hi_ref[...]], axis=1)
    h2 = jnp.maximum(v2 * dis + b2_ref[...], 0.0)
    u3 = h2 * dis
    ulo_ref[...] = u3[:, :HID // 2]
    uhi_ref[...] = u3[:, HID // 2:]


_tc_c = pl.pallas_call(
    _tc_c_body,
    grid=(NR // RB,),
    in_specs=[pl.BlockSpec((RB, HID // 2), lambda i: (i, 0)),
              pl.BlockSpec((RB, HID // 2), lambda i: (i, 0)),
              pl.BlockSpec((RB, HID // 2), lambda i: (i, 0)),
              pl.BlockSpec((RB, HID // 2), lambda i: (i, 0)),
              pl.BlockSpec((RB, IN_C), lambda i: (i, 0)),
              pl.BlockSpec((1, HID), lambda i: (0, 0))],
    out_specs=[pl.BlockSpec((RB, HID // 2), lambda i: (i, 0)),
               pl.BlockSpec((RB, HID // 2), lambda i: (i, 0))],
    out_shape=[jax.ShapeDtypeStruct((NR, HID // 2), _f32),
               jax.ShapeDtypeStruct((NR, HID // 2), _f32)],
)


def _tc_d_body(vlo_ref, vhi_ref, u3lo_ref, u3hi_ref, dis_ref, w3_ref,
               b3_ref, out_ref):
    dis = dis_ref[:, 0:1]
    v3 = jnp.concatenate([vlo_ref[...] + u3lo_ref[...],
                          vhi_ref[...] + u3hi_ref[...]], axis=1)
    z = jnp.dot(v3 * dis, w3_ref[...], preferred_element_type=_f32)
    z = z + b3_ref[...]
    m = jnp.max(z, axis=1, keepdims=True)
    e = jnp.exp(z - m)
    lse = jnp.log(jnp.sum(e, axis=1, keepdims=True)) + m
    out_ref[...] = z - lse


_tc_d = pl.pallas_call(
    _tc_d_body,
    grid=(NR // RB,),
    in_specs=[pl.BlockSpec((RB, HID // 2), lambda i: (i, 0)),
              pl.BlockSpec((RB, HID // 2), lambda i: (i, 0)),
              pl.BlockSpec((RB, HID // 2), lambda i: (i, 0)),
              pl.BlockSpec((RB, HID // 2), lambda i: (i, 0)),
              pl.BlockSpec((RB, IN_C), lambda i: (i, 0)),
              pl.BlockSpec((HID, OUT_C), lambda i: (0, 0)),
              pl.BlockSpec((1, OUT_C), lambda i: (0, 0))],
    out_specs=pl.BlockSpec((RB, OUT_C), lambda i: (i, 0)),
    out_shape=jax.ShapeDtypeStruct((NR, OUT_C), _f32),
)


def kernel(x, local_edges_list, remote_edges_list, W1, b1, W2, b2, W3, b3):
    src = jnp.concatenate([local_edges_list[0], remote_edges_list[0]])
    dst = jnp.concatenate([local_edges_list[1], remote_edges_list[1]])
    src = jnp.pad(src, (0, E_PAD - E_TOTAL), constant_values=N)
    dst = jnp.pad(dst, (0, E_PAD - E_TOTAL), constant_values=N)
    srcs4 = src.reshape(NTILES, NSLAB, SLAB, CHUNK)
    dsts4 = dst.reshape(NTILES, NSLAB, SLAB, CHUNK)
    srcs32 = src.reshape(NSLAB_E, 2 * NTILES, SLAB, CHUNK).transpose(
        1, 0, 2, 3)
    dsts32 = dst.reshape(NSLAB_E, 2 * NTILES, SLAB, CHUNK).transpose(
        1, 0, 2, 3)
    dsts32d = dst.reshape(2 * NTILES, HALF, CHUNK)

    xp = jnp.pad(x, ((0, NR - N), (0, 0)))
    b1r = b1.reshape(1, HID)
    b2r = b2.reshape(1, HID)
    b3r = b3.reshape(1, OUT_C)

    deg2 = _deg_sc(dsts32d)
    dis, u1a, u1b = _tc_a(deg2.T, xp)
    v1p = _agg_e(u1a, u1b, srcs32, dsts32)
    u2lo, u2hi = _tc_b(v1p, u1a, dis, W1, b1r, W2)
    v2lo, v2hi = _agg_c(u2lo, u2hi, srcs4, dsts4)
    u3lo, u3hi = _tc_c(v2lo, v2hi, u2lo, u2hi, dis, b2r)
    v3lo, v3hi = _agg_c(u3lo, u3hi, srcs4, dsts4)
    out = _tc_d(v3lo, v3hi, u3lo, u3hi, dis, W3, b3r)
    return out[:N]


# speedup vs baseline: 1.4395x; 1.0004x over previous
import functools

import jax
import jax.numpy as jnp
from jax import lax
from jax.experimental import pallas as pl
from jax.experimental.pallas import tpu as pltpu
from jax.experimental.pallas import tpu_sc as plsc

N = 10000
NR = 10240
IN_C = 128
HID = 256
OUT_C = 172
E_LOCAL = 300000
E_REMOTE = 20000
E_TOTAL = E_LOCAL + E_REMOTE
NTILES = 16
CHUNK = 64
NCHUNK = 320
SLAB = 16
NSLAB = NCHUNK // SLAB
E_PAD = NTILES * NCHUNK * CHUNK
RPT = NR // NTILES
RB = 512
HALF = NCHUNK // 2
NSLAB_E = HALF // SLAB

_f32 = jnp.float32
_mesh = plsc.VectorSubcoreMesh(core_axis_name="c", subcore_axis_name="s")
_params = pltpu.CompilerParams(needs_layout_passes=False)


@functools.partial(
    pl.kernel, mesh=_mesh,
    out_type=jax.ShapeDtypeStruct((2, NR), _f32),
    compiler_params=_params,
    scratch_types=[
        pltpu.VMEM((HALF, CHUNK), jnp.int32),
        pltpu.VMEM((NR,), _f32),
        pltpu.MemorySpace.VMEM_SHARED((NTILES, NR), _f32),
        pltpu.VMEM((NTILES, RPT), _f32),
        pltpu.VMEM((RPT,), _f32),
    ],
)
def _deg_sc(dsts32, deg_out, dstv, pdeg, stage, redbuf, res):
    c = lax.axis_index("c")
    s = lax.axis_index("s")
    pltpu.sync_copy(dsts32.at[c * NTILES + s], dstv)

    zeros16 = jnp.zeros((16,), _f32)
    ones16 = jnp.ones((16,), _f32)

    def _zero(i, _):
        pdeg[pl.ds(i * 16, 16)] = zeros16
        return ()
    lax.fori_loop(0, NR // 16, _zero, ())

    def _accum(k, _):
        r = k // (CHUNK // 16)
        i = k - r * (CHUNK // 16)
        idx = dstv[r, pl.ds(i * 16, 16)]
        plsc.addupdate_scatter(pdeg, [idx], ones16)
        return ()
    lax.fori_loop(0, HALF * (CHUNK // 16), _accum, ())

    pltpu.sync_copy(pdeg, stage.at[s])
    plsc.subcore_barrier()
    pltpu.sync_copy(stage.at[:, pl.ds(s * RPT, RPT)], redbuf)

    def _red(i, _):
        def _row(r, a):
            return a + redbuf[r, pl.ds(i * 16, 16)]
        res[pl.ds(i * 16, 16)] = lax.fori_loop(0, NTILES, _row, zeros16)
        return ()
    lax.fori_loop(0, RPT // 16, _red, ())
    pltpu.sync_copy(res, deg_out.at[c, pl.ds(s * RPT, RPT)])


def _edge_loop(g, acc, srcs4, dsts4, tid, srcv, dstv, rbs, sgs, sss, nslab):
    def slab_body(gs, _):
        pltpu.sync_copy(srcs4.at[tid, gs], srcv)
        pltpu.sync_copy(dsts4.at[tid, gs], dstv)
        for b in range(3):
            pltpu.async_copy(g.at[srcv.at[b]], rbs[b], sgs[b])

        def quad(i, _):
            j0 = 4 * i
            for b in range(4):
                j = j0 + b
                pltpu.make_async_copy(g.at[srcv.at[j]], rbs[b],
                                      sgs[b]).wait()
                pltpu.async_copy(rbs[b], acc.at[dstv.at[j]], sss[b],
                                 add=True)
                bn = (b + 3) % 4

                @pl.when(j + 3 < SLAB)
                def _():
                    @pl.when(j >= 1)
                    def _():
                        pltpu.make_async_copy(rbs[bn],
                                              acc.at[dstv.at[j - 1]],
                                              sss[bn]).wait()
                    pltpu.async_copy(g.at[srcv.at[j + 3]], rbs[bn],
                                     sgs[bn])
            return ()
        lax.fori_loop(0, SLAB // 4, quad, ())
        for b in range(4):
            pltpu.make_async_copy(rbs[b], acc.at[dstv.at[SLAB - 4 + b]],
                                  sss[b]).wait()
        return ()
    lax.fori_loop(0, nslab, slab_body, ())


def _zero_acc(acc, rb, s):
    zeros16 = jnp.zeros((16,), _f32)

    def _zb(i, _):
        r = i // (IN_C // 16)
        k = i - r * (IN_C // 16)
        rb[r, pl.ds(k * 16, 16)] = zeros16
        return ()
    lax.fori_loop(0, CHUNK * (IN_C // 16), _zb, ())

    def _cp(i, _):
        pltpu.sync_copy(rb, acc.at[pl.ds(s * RPT + i * CHUNK, CHUNK)])
        return ()
    lax.fori_loop(0, RPT // CHUNK, _cp, ())


@functools.partial(
    pl.kernel, mesh=_mesh,
    out_type=jax.ShapeDtypeStruct((2, NR, IN_C), _f32),
    compiler_params=_params,
    scratch_types=[
        pltpu.MemorySpace.VMEM_SHARED((NR, IN_C), _f32),
        pltpu.VMEM((SLAB, CHUNK), jnp.int32),
        pltpu.VMEM((SLAB, CHUNK), jnp.int32),
        pltpu.VMEM((CHUNK, IN_C), _f32),
        pltpu.VMEM((CHUNK, IN_C), _f32),
        pltpu.VMEM((CHUNK, IN_C), _f32),
        pltpu.VMEM((CHUNK, IN_C), _f32),
        pltpu.SemaphoreType.DMA,
        pltpu.SemaphoreType.DMA,
        pltpu.SemaphoreType.DMA,
        pltpu.SemaphoreType.DMA,
        pltpu.SemaphoreType.DMA,
        pltpu.SemaphoreType.DMA,
        pltpu.SemaphoreType.DMA,
        pltpu.SemaphoreType.DMA,
    ],
)
def _agg_e(g_a, g_b, srcs4_a, dsts4_a, srcs4_b, dsts4_b, out, acc,
           srcv, dstv, rb0, rb1, rb2, rb3,
           sg0, sg1, sg2, sg3, ss0, ss1, ss2, ss3):
    c = lax.axis_index("c")
    s = lax.axis_index("s")
    _zero_acc(acc, rb0, s)
    plsc.subcore_barrier()

    def run(g, srcs4, dsts4):
        _edge_loop(g, acc, srcs4, dsts4, c * NTILES + s, srcv, dstv,
                   (rb0, rb1, rb2, rb3), (sg0, sg1, sg2, sg3),
                   (ss0, ss1, ss2, ss3), NSLAB_E)

    @pl.when(c == 0)
    def _():
        run(g_a, srcs4_a, dsts4_a)

    @pl.when(c == 1)
    def _():
        run(g_b, srcs4_b, dsts4_b)
    plsc.subcore_barrier()
    pltpu.sync_copy(acc.at[pl.ds(s * RPT, RPT)],
                    out.at[c, pl.ds(s * RPT, RPT)])


@functools.partial(
    pl.kernel, mesh=_mesh,
    out_type=(jax.ShapeDtypeStruct((NR, IN_C), _f32),
              jax.ShapeDtypeStruct((NR, IN_C), _f32)),
    compiler_params=_params,
    scratch_types=[
        pltpu.MemorySpace.VMEM_SHARED((NR, IN_C), _f32),
        pltpu.VMEM((SLAB, CHUNK), jnp.int32),
        pltpu.VMEM((SLAB, CHUNK), jnp.int32),
        pltpu.VMEM((CHUNK, IN_C), _f32),
        pltpu.VMEM((CHUNK, IN_C), _f32),
        pltpu.VMEM((CHUNK, IN_C), _f32),
        pltpu.VMEM((CHUNK, IN_C), _f32),
        pltpu.SemaphoreType.DMA,
        pltpu.SemaphoreType.DMA,
        pltpu.SemaphoreType.DMA,
        pltpu.SemaphoreType.DMA,
        pltpu.SemaphoreType.DMA,
        pltpu.SemaphoreType.DMA,
        pltpu.SemaphoreType.DMA,
        pltpu.SemaphoreType.DMA,
    ],
)
def _agg_c(g_lo, g_hi, srcs4_a, dsts4_a, srcs4_b, dsts4_b, out_lo, out_hi,
           acc, srcv, dstv, rb0, rb1, rb2, rb3,
           sg0, sg1, sg2, sg3, ss0, ss1, ss2, ss3):
    c = lax.axis_index("c")
    s = lax.axis_index("s")
    _zero_acc(acc, rb0, s)
    plsc.subcore_barrier()

    def run(g, srcs4, dsts4, out):
        _edge_loop(g, acc, srcs4, dsts4, s, srcv, dstv,
                   (rb0, rb1, rb2, rb3), (sg0, sg1, sg2, sg3),
                   (ss0, ss1, ss2, ss3), NSLAB)
        plsc.subcore_barrier()
        pltpu.sync_copy(acc.at[pl.ds(s * RPT, RPT)],
                        out.at[pl.ds(s * RPT, RPT)])

    @pl.when(c == 0)
    def _():
        run(g_lo, srcs4_a, dsts4_a, out_lo)

    @pl.when(c == 1)
    def _():
        run(g_hi, srcs4_b, dsts4_b, out_hi)


def _tc_a_body(deg_ref, x_ref, dis_ref, u1a_ref, u1b_ref):
    i = pl.program_id(0)
    degsum = deg_ref[:, 0:1] + deg_ref[:, 1:2] + 1.0
    rows = i * RB + lax.broadcasted_iota(jnp.int32, (RB, 1), 0)
    dis = jnp.where(rows < N, lax.rsqrt(degsum), 0.0)
    dis_ref[...] = jnp.broadcast_to(dis, (RB, IN_C))
    u1 = x_ref[...] * dis
    u1a_ref[...] = u1
    u1b_ref[...] = u1


_tc_a = pl.pallas_call(
    _tc_a_body,
    grid=(NR // RB,),
    in_specs=[pl.BlockSpec((RB, 2), lambda i: (i, 0)),
              pl.BlockSpec((RB, IN_C), lambda i: (i, 0))],
    out_specs=[pl.BlockSpec((RB, IN_C), lambda i: (i, 0)),
               pl.BlockSpec((RB, IN_C), lambda i: (i, 0)),
               pl.BlockSpec((RB, IN_C), lambda i: (i, 0))],
    out_shape=[jax.ShapeDtypeStruct((NR, IN_C), _f32),
               jax.ShapeDtypeStruct((NR, IN_C), _f32),
               jax.ShapeDtypeStruct((NR, IN_C), _f32)],
)


def _tc_b_body(v1_ref, u1_ref, dis_ref, w1_ref, b1_ref, w2_ref,
               ulo_ref, uhi_ref):
    dis = dis_ref[:, 0:1]
    v1 = v1_ref[0] + v1_ref[1] + u1_ref[...]
    h1 = jnp.dot(v1 * dis, w1_ref[...], preferred_element_type=_f32)
    h1 = jnp.maximum(h1 + b1_ref[...], 0.0)
    p2 = jnp.dot(h1, w2_ref[...], preferred_element_type=_f32)
    u2 = p2 * dis
    ulo_ref[...] = u2[:, :HID // 2]
    uhi_ref[...] = u2[:, HID // 2:]


_tc_b = pl.pallas_call(
    _tc_b_body,
    grid=(NR // RB,),
    in_specs=[pl.BlockSpec((2, RB, IN_C), lambda i: (0, i, 0)),
              pl.BlockSpec((RB, IN_C), lambda i: (i, 0)),
              pl.BlockSpec((RB, IN_C), lambda i: (i, 0)),
              pl.BlockSpec((IN_C, HID), lambda i: (0, 0)),
              pl.BlockSpec((1, HID), lambda i: (0, 0)),
              pl.BlockSpec((HID, HID), lambda i: (0, 0))],
    out_specs=[pl.BlockSpec((RB, HID // 2), lambda i: (i, 0)),
               pl.BlockSpec((RB, HID // 2), lambda i: (i, 0))],
    out_shape=[jax.ShapeDtypeStruct((NR, HID // 2), _f32),
               jax.ShapeDtypeStruct((NR, HID // 2), _f32)],
)


def _tc_c_body(vlo_ref, vhi_ref, u2lo_ref, u2hi_ref, dis_ref, b2_ref,
               ulo_ref, uhi_ref):
    dis = dis_ref[:, 0:1]
    v2 = jnp.concatenate([vlo_ref[...] + u2lo_ref[...],
                          vhi_ref[...] + u2hi_ref[...]], axis=1)
    h2 = jnp.maximum(v2 * dis + b2_ref[...], 0.0)
    u3 = h2 * dis
    ulo_ref[...] = u3[:, :HID // 2]
    uhi_ref[...] = u3[:, HID // 2:]


_tc_c = pl.pallas_call(
    _tc_c_body,
    grid=(NR // RB,),
    in_specs=[pl.BlockSpec((RB, HID // 2), lambda i: (i, 0)),
              pl.BlockSpec((RB, HID // 2), lambda i: (i, 0)),
              pl.BlockSpec((RB, HID // 2), lambda i: (i, 0)),
              pl.BlockSpec((RB, HID // 2), lambda i: (i, 0)),
              pl.BlockSpec((RB, IN_C), lambda i: (i, 0)),
              pl.BlockSpec((1, HID), lambda i: (0, 0))],
    out_specs=[pl.BlockSpec((RB, HID // 2), lambda i: (i, 0)),
               pl.BlockSpec((RB, HID // 2), lambda i: (i, 0))],
    out_shape=[jax.ShapeDtypeStruct((NR, HID // 2), _f32),
               jax.ShapeDtypeStruct((NR, HID // 2), _f32)],
)


def _tc_d_body(vlo_ref, vhi_ref, u3lo_ref, u3hi_ref, dis_ref, w3_ref,
               b3_ref, out_ref):
    dis = dis_ref[:, 0:1]
    v3 = jnp.concatenate([vlo_ref[...] + u3lo_ref[...],
                          vhi_ref[...] + u3hi_ref[...]], axis=1)
    z = jnp.dot(v3 * dis, w3_ref[...], preferred_element_type=_f32)
    z = z + b3_ref[...]
    m = jnp.max(z, axis=1, keepdims=True)
    e = jnp.exp(z - m)
    lse = jnp.log(jnp.sum(e, axis=1, keepdims=True)) + m
    out_ref[...] = z - lse


_tc_d = pl.pallas_call(
    _tc_d_body,
    grid=(NR // RB,),
    in_specs=[pl.BlockSpec((RB, HID // 2), lambda i: (i, 0)),
              pl.BlockSpec((RB, HID // 2), lambda i: (i, 0)),
              pl.BlockSpec((RB, HID // 2), lambda i: (i, 0)),
              pl.BlockSpec((RB, HID // 2), lambda i: (i, 0)),
              pl.BlockSpec((RB, IN_C), lambda i: (i, 0)),
              pl.BlockSpec((HID, OUT_C), lambda i: (0, 0)),
              pl.BlockSpec((1, OUT_C), lambda i: (0, 0))],
    out_specs=pl.BlockSpec((RB, OUT_C), lambda i: (i, 0)),
    out_shape=jax.ShapeDtypeStruct((NR, OUT_C), _f32),
)


def kernel(x, local_edges_list, remote_edges_list, W1, b1, W2, b2, W3, b3):
    src = jnp.concatenate([local_edges_list[0], remote_edges_list[0]])
    dst = jnp.concatenate([local_edges_list[1], remote_edges_list[1]])
    src = jnp.pad(src, (0, E_PAD - E_TOTAL), constant_values=N)
    dst = jnp.pad(dst, (0, E_PAD - E_TOTAL), constant_values=N)
    srcs4a = src.reshape(NTILES, NSLAB, SLAB, CHUNK)
    dsts4a = dst.reshape(NTILES, NSLAB, SLAB, CHUNK)
    srcs4b = jnp.bitwise_or(srcs4a, 0)
    dsts4b = jnp.bitwise_or(dsts4a, 0)
    srcs32a = src.reshape(NSLAB_E, 2 * NTILES, SLAB, CHUNK).transpose(
        1, 0, 2, 3)
    dsts32a = dst.reshape(NSLAB_E, 2 * NTILES, SLAB, CHUNK).transpose(
        1, 0, 2, 3)
    srcs32b = jnp.bitwise_or(srcs32a, 0)
    dsts32b = jnp.bitwise_or(dsts32a, 0)
    dsts32d = dst.reshape(2 * NTILES, HALF, CHUNK)

    xp = jnp.pad(x, ((0, NR - N), (0, 0)))
    b1r = b1.reshape(1, HID)
    b2r = b2.reshape(1, HID)
    b3r = b3.reshape(1, OUT_C)

    deg2 = _deg_sc(dsts32d)
    dis, u1a, u1b = _tc_a(deg2.T, xp)
    v1p = _agg_e(u1a, u1b, srcs32a, dsts32a, srcs32b, dsts32b)
    u2lo, u2hi = _tc_b(v1p, u1a, dis, W1, b1r, W2)
    v2lo, v2hi = _agg_c(u2lo, u2hi, srcs4a, dsts4a, srcs4b, dsts4b)
    u3lo, u3hi = _tc_c(v2lo, v2hi, u2lo, u2hi, dis, b2r)
    v3lo, v3hi = _agg_c(u3lo, u3hi, srcs4a, dsts4a, srcs4b, dsts4b)
    out = _tc_d(v3lo, v3hi, u3lo, u3hi, dis, W3, b3r)
    return out[:N]
